# baseline (speedup 1.0000x reference)
import functools

import jax
import jax.numpy as jnp
from jax import lax
from jax.experimental import pallas as pl
from jax.experimental.pallas import tpu as pltpu
from jax.experimental.pallas import tpu_sc as plsc


@functools.lru_cache(maxsize=None)
def _make_gather(V, D, B):
    info = plsc.get_sparse_core_info()
    NC, NS = info.num_cores, info.num_subcores
    NW = NC * NS
    assert B % (8 * NW) == 0, (B, NW)
    b_per_w = B // NW
    C = 512
    while b_per_w % C:
        C //= 2
    n_chunks = b_per_w // C
    mesh = plsc.VectorSubcoreMesh(core_axis_name="c", subcore_axis_name="s")

    @functools.partial(
        pl.kernel,
        mesh=mesh,
        out_type=jax.ShapeDtypeStruct((B, D), jnp.float32),
        scratch_types=[
            pltpu.VMEM((C,), jnp.int32),
            pltpu.VMEM((C, D), jnp.float32),
            pltpu.SemaphoreType.DMA,
        ],
        compiler_params=pltpu.CompilerParams(use_tc_tiling_on_sc=False),
    )
    def k(idx_hbm, table_hbm, out_hbm, idx_v, rows_v, sem):
        wid = lax.axis_index("s") * NC + lax.axis_index("c")
        base = wid * b_per_w

        def step(i, carry):
            off = base + i * C
            pltpu.sync_copy(idx_hbm.at[pl.ds(off, C)], idx_v)
            pltpu.async_copy(table_hbm.at[idx_v], rows_v, sem).wait()
            pltpu.sync_copy(rows_v, out_hbm.at[pl.ds(off, C)])
            return carry

        lax.fori_loop(0, n_chunks, step, 0)

    return k


def kernel(index, cluster_index):
    B_rows, F = index.shape
    V, D = cluster_index.shape
    B = B_rows * F
    idx_flat = index.reshape(B)
    out = _make_gather(V, D, B)(idx_flat, cluster_index)
    return out.reshape(B_rows, F, D)


# speedup vs baseline: 1.0276x; 1.0276x over previous
import functools

import jax
import jax.numpy as jnp
from jax import lax
from jax.experimental import pallas as pl
from jax.experimental.pallas import tpu as pltpu
from jax.experimental.pallas import tpu_sc as plsc


@functools.lru_cache(maxsize=None)
def _make_gather(V, D, B):
    info = plsc.get_sparse_core_info()
    NC, NS = info.num_cores, info.num_subcores
    NW = NC * NS
    assert B % (8 * NW) == 0, (B, NW)
    b_per_w = B // NW
    C = 832
    while b_per_w % (2 * C):
        C //= 2
    n_pairs = b_per_w // (2 * C)
    mesh = plsc.VectorSubcoreMesh(core_axis_name="c", subcore_axis_name="s")

    @functools.partial(
        pl.kernel,
        mesh=mesh,
        out_type=jax.ShapeDtypeStruct((B, D), jnp.float32),
        scratch_types=[
            pltpu.VMEM((b_per_w,), jnp.int32),
            pltpu.VMEM((C, D), jnp.float32),
            pltpu.VMEM((C, D), jnp.float32),
            pltpu.SemaphoreType.DMA,
            pltpu.SemaphoreType.DMA,
            pltpu.SemaphoreType.DMA,
            pltpu.SemaphoreType.DMA,
        ],
        compiler_params=pltpu.CompilerParams(use_tc_tiling_on_sc=False),
    )
    def k(idx_hbm, table_hbm, out_hbm, idx_v, rows0, rows1, gs0, gs1, ws0, ws1):
        rows_b = (rows0, rows1)
        gs = (gs0, gs1)
        ws = (ws0, ws1)
        wid = lax.axis_index("s") * NC + lax.axis_index("c")
        base = wid * b_per_w

        pltpu.sync_copy(idx_hbm.at[pl.ds(base, b_per_w)], idx_v)

        def start_gather(g, b):
            pltpu.async_copy(
                table_hbm.at[idx_v.at[pl.ds(g * C, C)]], rows_b[b], gs[b]
            )

        def wait_gather(b):
            pltpu.make_async_copy(
                table_hbm.at[idx_v.at[pl.ds(0, C)]], rows_b[b], gs[b]
            ).wait()

        def start_write(g, b):
            pltpu.async_copy(rows_b[b], out_hbm.at[pl.ds(base + g * C, C)], ws[b])

        def wait_write(b):
            pltpu.make_async_copy(
                rows_b[b], out_hbm.at[pl.ds(base, C)], ws[b]
            ).wait()

        start_gather(0, 0)
        start_gather(1, 1)

        def pair(p, carry):
            g0 = 2 * p
            for b in range(2):
                g = g0 + b
                wait_gather(b)
                start_write(g, b)
                @pl.when(g + 2 < 2 * n_pairs)
                def _():
                    wait_write(b)
                    start_gather(g + 2, b)
            return carry

        lax.fori_loop(0, n_pairs, pair, 0)
        wait_write(0)
        wait_write(1)

    return k


def kernel(index, cluster_index):
    B_rows, F = index.shape
    V, D = cluster_index.shape
    B = B_rows * F
    idx_flat = index.reshape(B)
    out = _make_gather(V, D, B)(idx_flat, cluster_index)
    return out.reshape(B_rows, F, D)
